# baseline (speedup 1.0000x reference)
import functools

import jax
import jax.numpy as jnp
from jax import lax
from jax.experimental import pallas as pl
from jax.experimental.pallas import tpu as pltpu
from jax.experimental.pallas import tpu_sc as plsc

_B = 16384
_EMB = 64
_NC = 2
_NS = 16
_NW = _NC * _NS
_BPW = _B // _NW
_ICHUNK = 128
_NCH = _BPW // _ICHUNK


def _sc_body(uidx_hbm, bidx_hbm, uemb_hbm, ubias_hbm, bemb_hbm, bbias_hbm,
             part_out, bias_out,
             idx_u_v, idx_b_v, r_u_v, r_b_v, l_u_v, l_b_v,
             u_rows_v, b_rows_v, ub_rows_v, bb_rows_v, bias_v,
             acc_v, sem):
    wid = lax.axis_index("s") * _NC + lax.axis_index("c")
    base = wid * _BPW

    for c in range(_NCH):
        pltpu.sync_copy(uidx_hbm.at[pl.ds(base + c * _ICHUNK, _ICHUNK)],
                        idx_u_v.at[c])
        pltpu.sync_copy(bidx_hbm.at[pl.ds(base + c * _ICHUNK, _ICHUNK)],
                        idx_b_v.at[c])

    copies = []
    for c in range(_NCH):
        rows = pl.ds(c * _ICHUNK, _ICHUNK)
        copies.append(pltpu.async_copy(
            uemb_hbm.at[idx_u_v.at[c]], u_rows_v.at[rows, :], sem))
        copies.append(pltpu.async_copy(
            bemb_hbm.at[idx_b_v.at[c]], b_rows_v.at[rows, :], sem))

    for j in range(_BPW // 16):
        c, off = j // 8, (j % 8) * 16
        iu = idx_u_v[c, pl.ds(off, 16)]
        ib = idx_b_v[c, pl.ds(off, 16)]
        r_u_v[c, pl.ds(off, 16)] = lax.shift_right_logical(iu, 4)
        r_b_v[c, pl.ds(off, 16)] = lax.shift_right_logical(ib, 4)
        l_u_v[pl.ds(j * 16, 16)] = lax.bitwise_and(iu, 15)
        l_b_v[pl.ds(j * 16, 16)] = lax.bitwise_and(ib, 15)

    bias_copies = []
    for c in range(_NCH):
        rows = pl.ds(c * _ICHUNK, _ICHUNK)
        bias_copies.append(pltpu.async_copy(
            ubias_hbm.at[r_u_v.at[c]], ub_rows_v.at[rows, :], sem))
        bias_copies.append(pltpu.async_copy(
            bbias_hbm.at[r_b_v.at[c]], bb_rows_v.at[rows, :], sem))

    for cp in copies:
        cp.wait()

    zero = jnp.zeros((16,), jnp.float32)

    def body(i, accs):
        a0, a1, a2, a3 = accs
        a0 = a0 + u_rows_v[i, pl.ds(0, 16)] * b_rows_v[i, pl.ds(0, 16)]
        a1 = a1 + u_rows_v[i, pl.ds(16, 16)] * b_rows_v[i, pl.ds(16, 16)]
        a2 = a2 + u_rows_v[i, pl.ds(32, 16)] * b_rows_v[i, pl.ds(32, 16)]
        a3 = a3 + u_rows_v[i, pl.ds(48, 16)] * b_rows_v[i, pl.ds(48, 16)]
        return (a0, a1, a2, a3)

    a0, a1, a2, a3 = lax.fori_loop(0, _BPW, body, (zero, zero, zero, zero))
    acc_v[0, :] = (a0 + a1) + (a2 + a3)
    pltpu.sync_copy(acc_v, part_out.at[pl.ds(wid, 1), :])

    for cp in bias_copies:
        cp.wait()

    for j in range(_BPW // 16):
        e = lax.iota(jnp.int32, 16) + j * 16
        lu = l_u_v[pl.ds(j * 16, 16)]
        lb = l_b_v[pl.ds(j * 16, 16)]
        vu = plsc.load_gather(ub_rows_v, [e, lu])
        vb = plsc.load_gather(bb_rows_v, [e, lb])
        bias_v[pl.ds(j * 16, 16)] = vu + vb

    pltpu.sync_copy(bias_v, bias_out.at[pl.ds(base, _BPW)])


@functools.partial(
    pl.kernel,
    out_type=[
        jax.ShapeDtypeStruct((_NW, 16), jnp.float32),
        jax.ShapeDtypeStruct((_B,), jnp.float32),
    ],
    mesh=plsc.VectorSubcoreMesh(core_axis_name="c", subcore_axis_name="s"),
    compiler_params=pltpu.CompilerParams(use_tc_tiling_on_sc=False,
                                         needs_layout_passes=False),
    scratch_types=[
        pltpu.VMEM((_NCH, _ICHUNK), jnp.int32),
        pltpu.VMEM((_NCH, _ICHUNK), jnp.int32),
        pltpu.VMEM((_NCH, _ICHUNK), jnp.int32),
        pltpu.VMEM((_NCH, _ICHUNK), jnp.int32),
        pltpu.VMEM((_BPW,), jnp.int32),
        pltpu.VMEM((_BPW,), jnp.int32),
        pltpu.VMEM((_BPW, _EMB), jnp.float32),
        pltpu.VMEM((_BPW, _EMB), jnp.float32),
        pltpu.VMEM((_BPW, 16), jnp.float32),
        pltpu.VMEM((_BPW, 16), jnp.float32),
        pltpu.VMEM((_BPW,), jnp.float32),
        pltpu.VMEM((1, 16), jnp.float32),
        pltpu.SemaphoreType.DMA,
    ],
)
def _sc_call(uidx_hbm, bidx_hbm, uemb_hbm, ubias_hbm, bemb_hbm, bbias_hbm,
             part_out, bias_out, *scratch):
    _sc_body(uidx_hbm, bidx_hbm, uemb_hbm, ubias_hbm, bemb_hbm, bbias_hbm,
             part_out, bias_out, *scratch)


def _tc_body(part_ref, bias_ref, out_ref):
    s = jnp.sum(part_ref[...])
    out_ref[...] = jax.nn.sigmoid(bias_ref[...] + s)


_tc_call = pl.pallas_call(
    _tc_body,
    out_shape=jax.ShapeDtypeStruct((128, 128), jnp.float32),
)


def kernel(inputs, user_embedding, user_bias, book_embedding, book_bias):
    u_idx = inputs[:, 0]
    b_idx = inputs[:, 1]
    part, bias = _sc_call(u_idx, b_idx, user_embedding,
                          user_bias.reshape(-1, 16),
                          book_embedding, book_bias.reshape(-1, 16))
    y = _tc_call(part, bias.reshape(128, 128))
    return y.reshape(_B, 1)


# speedup vs baseline: 1.0638x; 1.0638x over previous
import functools

import jax
import jax.numpy as jnp
from jax import lax
from jax.experimental import pallas as pl
from jax.experimental.pallas import tpu as pltpu
from jax.experimental.pallas import tpu_sc as plsc

_B = 16384
_EMB = 64
_NC = 2
_NS = 16
_NW = _NC * _NS
_BPW = _B // _NW
_ICHUNK = 128
_NCH = _BPW // _ICHUNK


def _sc_body(uidx_hbm, bidx_hbm, uemb_hbm, ubias_hbm, bemb_hbm, bbias_hbm,
             part_out, bias_out,
             idx_u_v, idx_b_v, r_u_v, r_b_v, l_u_v, l_b_v,
             u_rows0_v, u_rows1_v, b_rows0_v, b_rows1_v,
             ub_rows_v, bb_rows_v, bias_v,
             acc_v, sem0, sem1, sem_bias):
    wid = lax.axis_index("s") * _NC + lax.axis_index("c")
    base = wid * _BPW

    for c in range(_NCH):
        pltpu.sync_copy(uidx_hbm.at[pl.ds(base + c * _ICHUNK, _ICHUNK)],
                        idx_u_v.at[c])
        pltpu.sync_copy(bidx_hbm.at[pl.ds(base + c * _ICHUNK, _ICHUNK)],
                        idx_b_v.at[c])

    for j in range(_BPW // 16):
        c, off = j // 8, (j % 8) * 16
        iu = idx_u_v[c, pl.ds(off, 16)]
        ib = idx_b_v[c, pl.ds(off, 16)]
        r_u_v[c, pl.ds(off, 16)] = lax.shift_right_logical(iu, 4)
        r_b_v[c, pl.ds(off, 16)] = lax.shift_right_logical(ib, 4)
        l_u_v[pl.ds(j * 16, 16)] = lax.bitwise_and(iu, 15)
        l_b_v[pl.ds(j * 16, 16)] = lax.bitwise_and(ib, 15)

    bias_copies = []
    for c in range(_NCH):
        rows = pl.ds(c * _ICHUNK, _ICHUNK)
        bias_copies.append(pltpu.async_copy(
            ubias_hbm.at[r_u_v.at[c]], ub_rows_v.at[rows, :], sem_bias))
        bias_copies.append(pltpu.async_copy(
            bbias_hbm.at[r_b_v.at[c]], bb_rows_v.at[rows, :], sem_bias))

    zero = jnp.zeros((16,), jnp.float32)
    u_bufs = (u_rows0_v, u_rows1_v)
    b_bufs = (b_rows0_v, b_rows1_v)

    sems = (sem0, sem1)

    def fire(c):
        return (pltpu.async_copy(uemb_hbm.at[idx_u_v.at[c]],
                                 u_bufs[c % 2], sems[c % 2]),
                pltpu.async_copy(bemb_hbm.at[idx_b_v.at[c]],
                                 b_bufs[c % 2], sems[c % 2]))

    pend = fire(0)
    accs = (zero, zero, zero, zero)
    for c in range(_NCH):
        for cp in pend:
            cp.wait()
        if c + 1 < _NCH:
            pend = fire(c + 1)
        u_ref, b_ref = u_bufs[c % 2], b_bufs[c % 2]

        def body(i, accs, u_ref=u_ref, b_ref=b_ref):
            a0, a1, a2, a3 = accs
            a0 = a0 + u_ref[i, pl.ds(0, 16)] * b_ref[i, pl.ds(0, 16)]
            a1 = a1 + u_ref[i, pl.ds(16, 16)] * b_ref[i, pl.ds(16, 16)]
            a2 = a2 + u_ref[i, pl.ds(32, 16)] * b_ref[i, pl.ds(32, 16)]
            a3 = a3 + u_ref[i, pl.ds(48, 16)] * b_ref[i, pl.ds(48, 16)]
            return (a0, a1, a2, a3)

        accs = lax.fori_loop(0, _ICHUNK, body, accs)

    a0, a1, a2, a3 = accs
    acc_v[0, :] = (a0 + a1) + (a2 + a3)
    pltpu.sync_copy(acc_v, part_out.at[pl.ds(wid, 1), :])

    for cp in bias_copies:
        cp.wait()

    for j in range(_BPW // 16):
        e = lax.iota(jnp.int32, 16) + j * 16
        lu = l_u_v[pl.ds(j * 16, 16)]
        lb = l_b_v[pl.ds(j * 16, 16)]
        vu = plsc.load_gather(ub_rows_v, [e, lu])
        vb = plsc.load_gather(bb_rows_v, [e, lb])
        bias_v[pl.ds(j * 16, 16)] = vu + vb

    pltpu.sync_copy(bias_v, bias_out.at[pl.ds(base, _BPW)])


@functools.partial(
    pl.kernel,
    out_type=[
        jax.ShapeDtypeStruct((_NW, 16), jnp.float32),
        jax.ShapeDtypeStruct((_B,), jnp.float32),
    ],
    mesh=plsc.VectorSubcoreMesh(core_axis_name="c", subcore_axis_name="s"),
    compiler_params=pltpu.CompilerParams(use_tc_tiling_on_sc=False,
                                         needs_layout_passes=False),
    scratch_types=[
        pltpu.VMEM((_NCH, _ICHUNK), jnp.int32),
        pltpu.VMEM((_NCH, _ICHUNK), jnp.int32),
        pltpu.VMEM((_NCH, _ICHUNK), jnp.int32),
        pltpu.VMEM((_NCH, _ICHUNK), jnp.int32),
        pltpu.VMEM((_BPW,), jnp.int32),
        pltpu.VMEM((_BPW,), jnp.int32),
        pltpu.VMEM((_ICHUNK, 128), jnp.float32),
        pltpu.VMEM((_ICHUNK, 128), jnp.float32),
        pltpu.VMEM((_ICHUNK, 128), jnp.float32),
        pltpu.VMEM((_ICHUNK, 128), jnp.float32),
        pltpu.VMEM((_BPW, 16), jnp.float32),
        pltpu.VMEM((_BPW, 16), jnp.float32),
        pltpu.VMEM((_BPW,), jnp.float32),
        pltpu.VMEM((1, 16), jnp.float32),
        pltpu.SemaphoreType.DMA,
        pltpu.SemaphoreType.DMA,
        pltpu.SemaphoreType.DMA,
    ],
)
def _sc_call(uidx_hbm, bidx_hbm, uemb_hbm, ubias_hbm, bemb_hbm, bbias_hbm,
             part_out, bias_out, *scratch):
    _sc_body(uidx_hbm, bidx_hbm, uemb_hbm, ubias_hbm, bemb_hbm, bbias_hbm,
             part_out, bias_out, *scratch)


def _tc_body(part_ref, bias_ref, out_ref):
    s = jnp.sum(part_ref[...])
    out_ref[...] = jax.nn.sigmoid(bias_ref[...] + s)


_tc_call = pl.pallas_call(
    _tc_body,
    out_shape=jax.ShapeDtypeStruct((128, 128), jnp.float32),
)


def kernel(inputs, user_embedding, user_bias, book_embedding, book_bias):
    u_idx = inputs[:, 0]
    b_idx = inputs[:, 1]
    uemb = jnp.pad(user_embedding, ((0, 0), (0, 64)))
    bemb = jnp.pad(book_embedding, ((0, 0), (0, 64)))
    part, bias = _sc_call(u_idx, b_idx, uemb,
                          user_bias.reshape(-1, 16),
                          bemb, book_bias.reshape(-1, 16))
    y = _tc_call(part, bias.reshape(128, 128))
    return y.reshape(_B, 1)


# speedup vs baseline: 2.2319x; 2.0980x over previous
import functools

import jax
import jax.numpy as jnp
from jax import lax
from jax.experimental import pallas as pl
from jax.experimental.pallas import tpu as pltpu
from jax.experimental.pallas import tpu_sc as plsc

_B = 16384
_EMB = 64
_NC = 2
_NS = 16
_NW = _NC * _NS
_BPW = _B // _NW
_ICHUNK = 128
_NCH = _BPW // _ICHUNK
_RING = 4
_NGRP = _BPW // _RING


def _emb_body(uidx_hbm, bidx_hbm, uemb_t, bemb_t, part_out,
              idx_u_v, idx_b_v, u_bufs, b_bufs, acc_v, usems, bsems):
    wid = lax.axis_index("s") * _NC + lax.axis_index("c")
    base = wid * _BPW

    for c in range(_NCH):
        pltpu.sync_copy(uidx_hbm.at[pl.ds(base + c * _ICHUNK, _ICHUNK)],
                        idx_u_v.at[pl.ds(c * _ICHUNK, _ICHUNK)])
        pltpu.sync_copy(bidx_hbm.at[pl.ds(base + c * _ICHUNK, _ICHUNK)],
                        idx_b_v.at[pl.ds(c * _ICHUNK, _ICHUNK)])

    def getid(idx_ref, e):
        return idx_ref[pl.ds(e, 16)][0]

    def win(idx_ref, e):
        i = getid(idx_ref, e)
        return pl.multiple_of((i >> 7) << 7, 128)

    def fire(g, b):
        e = g * _RING + b
        pltpu.async_copy(uemb_t.at[:, pl.ds(win(idx_u_v, e), 128)],
                         u_bufs[b], usems[b])
        pltpu.async_copy(bemb_t.at[:, pl.ds(win(idx_b_v, e), 128)],
                         b_bufs[b], bsems[b])

    for b in range(_RING):
        fire(0, b)

    zero = jnp.zeros((16,), jnp.float32)
    jvecs = [lax.iota(jnp.int32, 16) + 16 * c for c in range(4)]

    def group(g, accs):
        accs = list(accs)
        for b in range(_RING):
            pltpu.make_async_copy(uemb_t.at[:, pl.ds(0, 128)],
                                  u_bufs[b], usems[b]).wait()
            pltpu.make_async_copy(bemb_t.at[:, pl.ds(0, 128)],
                                  b_bufs[b], bsems[b]).wait()
            e = g * _RING + b
            lu = jnp.broadcast_to(getid(idx_u_v, e) & 127, (16,))
            lb = jnp.broadcast_to(getid(idx_b_v, e) & 127, (16,))
            for c in range(4):
                uv = plsc.load_gather(u_bufs[b], [jvecs[c], lu])
                bv = plsc.load_gather(b_bufs[b], [jvecs[c], lb])
                accs[c] = accs[c] + uv * bv

            @pl.when(g < _NGRP - 1)
            def _():
                fire(g + 1, b)

        return tuple(accs)

    a0, a1, a2, a3 = lax.fori_loop(0, _NGRP, group,
                                   (zero, zero, zero, zero))
    acc_v[0, :] = (a0 + a1) + (a2 + a3)
    pltpu.sync_copy(acc_v, part_out.at[pl.ds(wid, 1), :])


@functools.partial(
    pl.kernel,
    out_type=jax.ShapeDtypeStruct((_NW, 16), jnp.float32),
    mesh=plsc.VectorSubcoreMesh(core_axis_name="c", subcore_axis_name="s"),
    compiler_params=pltpu.CompilerParams(use_tc_tiling_on_sc=True,
                                         needs_layout_passes=False),
    scratch_types=(
        [pltpu.VMEM((_BPW + 16,), jnp.int32)] * 2
        + [pltpu.VMEM((_EMB, 128), jnp.float32)] * (2 * _RING)
        + [pltpu.VMEM((1, 16), jnp.float32)]
        + [pltpu.SemaphoreType.DMA] * (2 * _RING)
    ),
)
def _emb_call(uidx_hbm, bidx_hbm, uemb_t, bemb_t, part_out, *scratch):
    idx_u_v, idx_b_v = scratch[0], scratch[1]
    u_bufs = scratch[2:2 + _RING]
    b_bufs = scratch[2 + _RING:2 + 2 * _RING]
    acc_v = scratch[2 + 2 * _RING]
    usems = scratch[3 + 2 * _RING:3 + 3 * _RING]
    bsems = scratch[3 + 3 * _RING:3 + 4 * _RING]
    _emb_body(uidx_hbm, bidx_hbm, uemb_t, bemb_t, part_out,
              idx_u_v, idx_b_v, u_bufs, b_bufs, acc_v, usems, bsems)


def _bias_body(uidx_hbm, bidx_hbm, ubias_hbm, bbias_hbm, bias_out,
               idx_u_v, idx_b_v, r_u_v, r_b_v, l_u_v, l_b_v,
               ub_rows_v, bb_rows_v, bias_v, sem):
    wid = lax.axis_index("s") * _NC + lax.axis_index("c")
    base = wid * _BPW

    for c in range(_NCH):
        pltpu.sync_copy(uidx_hbm.at[pl.ds(base + c * _ICHUNK, _ICHUNK)],
                        idx_u_v.at[c])
        pltpu.sync_copy(bidx_hbm.at[pl.ds(base + c * _ICHUNK, _ICHUNK)],
                        idx_b_v.at[c])

    for j in range(_BPW // 16):
        c, off = j // 8, (j % 8) * 16
        iu = idx_u_v[c, pl.ds(off, 16)]
        ib = idx_b_v[c, pl.ds(off, 16)]
        r_u_v[c, pl.ds(off, 16)] = lax.shift_right_logical(iu, 4)
        r_b_v[c, pl.ds(off, 16)] = lax.shift_right_logical(ib, 4)
        l_u_v[pl.ds(j * 16, 16)] = lax.bitwise_and(iu, 15)
        l_b_v[pl.ds(j * 16, 16)] = lax.bitwise_and(ib, 15)

    copies = []
    for c in range(_NCH):
        rows = pl.ds(c * _ICHUNK, _ICHUNK)
        copies.append(pltpu.async_copy(
            ubias_hbm.at[r_u_v.at[c]], ub_rows_v.at[rows, :], sem))
        copies.append(pltpu.async_copy(
            bbias_hbm.at[r_b_v.at[c]], bb_rows_v.at[rows, :], sem))
    for cp in copies:
        cp.wait()

    for j in range(_BPW // 16):
        e = lax.iota(jnp.int32, 16) + j * 16
        lu = l_u_v[pl.ds(j * 16, 16)]
        lb = l_b_v[pl.ds(j * 16, 16)]
        vu = plsc.load_gather(ub_rows_v, [e, lu])
        vb = plsc.load_gather(bb_rows_v, [e, lb])
        bias_v[pl.ds(j * 16, 16)] = vu + vb

    pltpu.sync_copy(bias_v, bias_out.at[pl.ds(base, _BPW)])


@functools.partial(
    pl.kernel,
    out_type=jax.ShapeDtypeStruct((_B,), jnp.float32),
    mesh=plsc.VectorSubcoreMesh(core_axis_name="c", subcore_axis_name="s"),
    compiler_params=pltpu.CompilerParams(use_tc_tiling_on_sc=False,
                                         needs_layout_passes=False),
    scratch_types=[
        pltpu.VMEM((_NCH, _ICHUNK), jnp.int32),
        pltpu.VMEM((_NCH, _ICHUNK), jnp.int32),
        pltpu.VMEM((_NCH, _ICHUNK), jnp.int32),
        pltpu.VMEM((_NCH, _ICHUNK), jnp.int32),
        pltpu.VMEM((_BPW,), jnp.int32),
        pltpu.VMEM((_BPW,), jnp.int32),
        pltpu.VMEM((_BPW, 16), jnp.float32),
        pltpu.VMEM((_BPW, 16), jnp.float32),
        pltpu.VMEM((_BPW,), jnp.float32),
        pltpu.SemaphoreType.DMA,
    ],
)
def _bias_call(uidx_hbm, bidx_hbm, ubias_hbm, bbias_hbm, bias_out, *scratch):
    _bias_body(uidx_hbm, bidx_hbm, ubias_hbm, bbias_hbm, bias_out, *scratch)


def _tc_body(part_ref, bias_ref, out_ref):
    s = jnp.sum(part_ref[...])
    out_ref[...] = jax.nn.sigmoid(bias_ref[...] + s)


_tc_call = pl.pallas_call(
    _tc_body,
    out_shape=jax.ShapeDtypeStruct((128, 128), jnp.float32),
)


def kernel(inputs, user_embedding, user_bias, book_embedding, book_bias):
    u_idx = inputs[:, 0]
    b_idx = inputs[:, 1]
    part = _emb_call(u_idx, b_idx, user_embedding.T, book_embedding.T)
    bias = _bias_call(u_idx, b_idx, user_bias.reshape(-1, 16),
                      book_bias.reshape(-1, 16))
    y = _tc_call(part, bias.reshape(128, 128))
    return y.reshape(_B, 1)


# speedup vs baseline: 2.4193x; 1.0840x over previous
import functools

import jax
import jax.numpy as jnp
from jax import lax
from jax.experimental import pallas as pl
from jax.experimental.pallas import tpu as pltpu
from jax.experimental.pallas import tpu_sc as plsc

_B = 16384
_EMB = 64
_NC = 2
_NS = 16
_NW = _NC * _NS
_BPW = _B // _NW
_ICHUNK = 128
_NCH = _BPW // _ICHUNK
_RING = 4
_NGRP = _BPW // _RING


def _emb_body(uidx_hbm, bidx_hbm, uemb_t, bemb_t, ubias_t, bbias_t,
              part_out, bias_out,
              idx_u_v, idx_b_v, u_bufs, b_bufs, ub_bufs, bb_bufs,
              bias_v, acc_v, usems, bsems, ubsems, bbsems):
    wid = lax.axis_index("s") * _NC + lax.axis_index("c")
    base = wid * _BPW

    for c in range(_NCH):
        pltpu.sync_copy(uidx_hbm.at[pl.ds(base + c * _ICHUNK, _ICHUNK)],
                        idx_u_v.at[pl.ds(c * _ICHUNK, _ICHUNK)])
        pltpu.sync_copy(bidx_hbm.at[pl.ds(base + c * _ICHUNK, _ICHUNK)],
                        idx_b_v.at[pl.ds(c * _ICHUNK, _ICHUNK)])

    def getid(idx_ref, e):
        return idx_ref[pl.ds(e, 16)][0]

    def win(idx_ref, e):
        i = getid(idx_ref, e)
        return pl.multiple_of((i >> 7) << 7, 128)

    def fire(g, b):
        e = g * _RING + b
        wu, wb = win(idx_u_v, e), win(idx_b_v, e)
        pltpu.async_copy(uemb_t.at[:, pl.ds(wu, 128)], u_bufs[b], usems[b])
        pltpu.async_copy(bemb_t.at[:, pl.ds(wb, 128)], b_bufs[b], bsems[b])
        pltpu.async_copy(ubias_t.at[:, pl.ds(wu, 128)], ub_bufs[b], ubsems[b])
        pltpu.async_copy(bbias_t.at[:, pl.ds(wb, 128)], bb_bufs[b], bbsems[b])

    for b in range(_RING):
        fire(0, b)

    zero = jnp.zeros((16,), jnp.float32)
    zrow = jnp.zeros((16,), jnp.int32)
    jvecs = [lax.iota(jnp.int32, 16) + 16 * c for c in range(4)]

    def group(g, accs):
        accs = list(accs)
        for b in range(_RING):
            pltpu.make_async_copy(uemb_t.at[:, pl.ds(0, 128)],
                                  u_bufs[b], usems[b]).wait()
            pltpu.make_async_copy(bemb_t.at[:, pl.ds(0, 128)],
                                  b_bufs[b], bsems[b]).wait()
            pltpu.make_async_copy(ubias_t.at[:, pl.ds(0, 128)],
                                  ub_bufs[b], ubsems[b]).wait()
            pltpu.make_async_copy(bbias_t.at[:, pl.ds(0, 128)],
                                  bb_bufs[b], bbsems[b]).wait()
            e = g * _RING + b
            lu = jnp.broadcast_to(getid(idx_u_v, e) & 127, (16,))
            lb = jnp.broadcast_to(getid(idx_b_v, e) & 127, (16,))
            for c in range(4):
                uv = plsc.load_gather(u_bufs[b], [jvecs[c], lu])
                bv = plsc.load_gather(b_bufs[b], [jvecs[c], lb])
                accs[c] = accs[c] + uv * bv
            bu = plsc.load_gather(ub_bufs[b], [zrow, lu])
            bb2 = plsc.load_gather(bb_bufs[b], [zrow, lb])
            plsc.store_scatter(bias_v, [jnp.broadcast_to(e, (16,))],
                               bu + bb2)

            @pl.when(g < _NGRP - 1)
            def _():
                fire(g + 1, b)

        return tuple(accs)

    a0, a1, a2, a3 = lax.fori_loop(0, _NGRP, group,
                                   (zero, zero, zero, zero))
    acc_v[0, :] = (a0 + a1) + (a2 + a3)
    pltpu.sync_copy(acc_v, part_out.at[pl.ds(wid, 1), :])
    pltpu.sync_copy(bias_v, bias_out.at[pl.ds(base, _BPW)])


@functools.partial(
    pl.kernel,
    out_type=[jax.ShapeDtypeStruct((_NW, 16), jnp.float32),
              jax.ShapeDtypeStruct((_B,), jnp.float32)],
    mesh=plsc.VectorSubcoreMesh(core_axis_name="c", subcore_axis_name="s"),
    compiler_params=pltpu.CompilerParams(use_tc_tiling_on_sc=True,
                                         needs_layout_passes=False),
    scratch_types=(
        [pltpu.VMEM((_BPW + 16,), jnp.int32)] * 2
        + [pltpu.VMEM((_EMB, 128), jnp.float32)] * (2 * _RING)
        + [pltpu.VMEM((1, 128), jnp.float32)] * (2 * _RING)
        + [pltpu.VMEM((_BPW,), jnp.float32)]
        + [pltpu.VMEM((1, 16), jnp.float32)]
        + [pltpu.SemaphoreType.DMA] * (4 * _RING)
    ),
)
def _emb_call(uidx_hbm, bidx_hbm, uemb_t, bemb_t, ubias_t, bbias_t,
              part_out, bias_out, *scratch):
    k = 2
    idx_u_v, idx_b_v = scratch[0], scratch[1]
    u_bufs = scratch[k:k + _RING]
    b_bufs = scratch[k + _RING:k + 2 * _RING]
    ub_bufs = scratch[k + 2 * _RING:k + 3 * _RING]
    bb_bufs = scratch[k + 3 * _RING:k + 4 * _RING]
    bias_v = scratch[k + 4 * _RING]
    acc_v = scratch[k + 4 * _RING + 1]
    sems = scratch[k + 4 * _RING + 2:]
    usems = sems[0:_RING]
    bsems = sems[_RING:2 * _RING]
    ubsems = sems[2 * _RING:3 * _RING]
    bbsems = sems[3 * _RING:4 * _RING]
    _emb_body(uidx_hbm, bidx_hbm, uemb_t, bemb_t, ubias_t, bbias_t,
              part_out, bias_out,
              idx_u_v, idx_b_v, u_bufs, b_bufs, ub_bufs, bb_bufs,
              bias_v, acc_v, usems, bsems, ubsems, bbsems)


def _bias_body(uidx_hbm, bidx_hbm, ubias_hbm, bbias_hbm, bias_out,
               idx_u_v, idx_b_v, r_u_v, r_b_v, l_u_v, l_b_v,
               ub_rows_v, bb_rows_v, bias_v, sem):
    wid = lax.axis_index("s") * _NC + lax.axis_index("c")
    base = wid * _BPW

    for c in range(_NCH):
        pltpu.sync_copy(uidx_hbm.at[pl.ds(base + c * _ICHUNK, _ICHUNK)],
                        idx_u_v.at[c])
        pltpu.sync_copy(bidx_hbm.at[pl.ds(base + c * _ICHUNK, _ICHUNK)],
                        idx_b_v.at[c])

    for j in range(_BPW // 16):
        c, off = j // 8, (j % 8) * 16
        iu = idx_u_v[c, pl.ds(off, 16)]
        ib = idx_b_v[c, pl.ds(off, 16)]
        r_u_v[c, pl.ds(off, 16)] = lax.shift_right_logical(iu, 4)
        r_b_v[c, pl.ds(off, 16)] = lax.shift_right_logical(ib, 4)
        l_u_v[pl.ds(j * 16, 16)] = lax.bitwise_and(iu, 15)
        l_b_v[pl.ds(j * 16, 16)] = lax.bitwise_and(ib, 15)

    copies = []
    for c in range(_NCH):
        rows = pl.ds(c * _ICHUNK, _ICHUNK)
        copies.append(pltpu.async_copy(
            ubias_hbm.at[r_u_v.at[c]], ub_rows_v.at[rows, :], sem))
        copies.append(pltpu.async_copy(
            bbias_hbm.at[r_b_v.at[c]], bb_rows_v.at[rows, :], sem))
    for cp in copies:
        cp.wait()

    for j in range(_BPW // 16):
        e = lax.iota(jnp.int32, 16) + j * 16
        lu = l_u_v[pl.ds(j * 16, 16)]
        lb = l_b_v[pl.ds(j * 16, 16)]
        vu = plsc.load_gather(ub_rows_v, [e, lu])
        vb = plsc.load_gather(bb_rows_v, [e, lb])
        bias_v[pl.ds(j * 16, 16)] = vu + vb

    pltpu.sync_copy(bias_v, bias_out.at[pl.ds(base, _BPW)])


@functools.partial(
    pl.kernel,
    out_type=jax.ShapeDtypeStruct((_B,), jnp.float32),
    mesh=plsc.VectorSubcoreMesh(core_axis_name="c", subcore_axis_name="s"),
    compiler_params=pltpu.CompilerParams(use_tc_tiling_on_sc=False,
                                         needs_layout_passes=False),
    scratch_types=[
        pltpu.VMEM((_NCH, _ICHUNK), jnp.int32),
        pltpu.VMEM((_NCH, _ICHUNK), jnp.int32),
        pltpu.VMEM((_NCH, _ICHUNK), jnp.int32),
        pltpu.VMEM((_NCH, _ICHUNK), jnp.int32),
        pltpu.VMEM((_BPW,), jnp.int32),
        pltpu.VMEM((_BPW,), jnp.int32),
        pltpu.VMEM((_BPW, 16), jnp.float32),
        pltpu.VMEM((_BPW, 16), jnp.float32),
        pltpu.VMEM((_BPW,), jnp.float32),
        pltpu.SemaphoreType.DMA,
    ],
)
def _bias_call(uidx_hbm, bidx_hbm, ubias_hbm, bbias_hbm, bias_out, *scratch):
    _bias_body(uidx_hbm, bidx_hbm, ubias_hbm, bbias_hbm, bias_out, *scratch)


def _tc_body(part_ref, bias_ref, out_ref):
    s = jnp.sum(part_ref[...])
    out_ref[...] = jax.nn.sigmoid(bias_ref[...] + s)


_tc_call = pl.pallas_call(
    _tc_body,
    out_shape=jax.ShapeDtypeStruct((128, 128), jnp.float32),
)


def kernel(inputs, user_embedding, user_bias, book_embedding, book_bias):
    u_idx = inputs[:, 0]
    b_idx = inputs[:, 1]
    part, bias = _emb_call(u_idx, b_idx, user_embedding.T, book_embedding.T,
                           user_bias.T, book_bias.T)
    y = _tc_call(part, bias.reshape(128, 128))
    return y.reshape(_B, 1)
